# baseline (speedup 1.0000x reference)
import functools

import jax
import jax.numpy as jnp
from jax import lax
from jax.experimental import pallas as pl
from jax.experimental.pallas import tpu as pltpu
from jax.experimental.pallas import tpu_sc as plsc

E = 64
K = 2
T = 2048
D = 768
F = 768
A = T * K
BT = 128
NT = A // BT + E
GT = 3
NTG = NT // GT
PR = NT * BT

NC = 2
NS = 16
NW = NC * NS


def _gate_body(x_ref, wg_ref, i1_ref, i2_ref, p1_ref, p2_ref):
    s = lax.dot_general(x_ref[...], wg_ref[...], (((1,), (1,)), ((), ())),
                        preferred_element_type=jnp.float32)
    bg = s.shape[0]
    lane = lax.broadcasted_iota(jnp.int32, (bg, E), 1)
    m1 = jnp.max(s, axis=1, keepdims=True)
    i1 = jnp.min(jnp.where(s >= m1, lane, E), axis=1, keepdims=True)
    s2 = jnp.where(lane == i1, -jnp.inf, s)
    m2 = jnp.max(s2, axis=1, keepdims=True)
    i2 = jnp.min(jnp.where(s2 >= m2, lane, E), axis=1, keepdims=True)
    p1 = jax.nn.sigmoid(m1 - m2)
    i1_ref[...] = i1
    i2_ref[...] = i2
    p1_ref[...] = p1
    p2_ref[...] = 1.0 - p1


def _gate(x_flat, Wg):
    BG = 256
    out = jax.ShapeDtypeStruct((T, 1), jnp.int32)
    outf = jax.ShapeDtypeStruct((T, 1), jnp.float32)
    return pl.pallas_call(
        _gate_body,
        grid=(T // BG,),
        in_specs=[
            pl.BlockSpec((BG, D), lambda t: (t, 0)),
            pl.BlockSpec((E, D), lambda t: (0, 0)),
        ],
        out_specs=[pl.BlockSpec((BG, 1), lambda t: (t, 0))] * 4,
        out_shape=[out, out, outf, outf],
    )(x_flat, Wg)


def _sc_gather_body(rw, c, table_hbm, idx_hbm, out_hbm, idx_v, rows_v, sem):
    wid = lax.axis_index("s") * NC + lax.axis_index("c")
    base = wid * rw

    def chunk(j, _):
        off = base + j * c
        pltpu.sync_copy(idx_hbm.at[pl.ds(off, c)], idx_v)
        pltpu.async_copy(table_hbm.at[idx_v], rows_v, sem).wait()
        pltpu.sync_copy(rows_v, out_hbm.at[pl.ds(off, c)])
        return 0

    lax.fori_loop(0, rw // c, chunk, 0)


def _sc_gather(table, idx, n_rows, chunk=128):
    rw = n_rows // NW
    mesh = plsc.VectorSubcoreMesh(core_axis_name="c", subcore_axis_name="s")
    kern = pl.kernel(
        functools.partial(_sc_gather_body, rw, chunk),
        out_type=jax.ShapeDtypeStruct((n_rows, D), jnp.float32),
        mesh=mesh,
        scratch_types=[
            pltpu.VMEM((chunk,), jnp.int32),
            pltpu.VMEM((chunk, D), jnp.float32),
            pltpu.SemaphoreType.DMA,
        ],
    )
    return kern(table, idx)


def _swiglu_tile(xb, wu, wv, wo):
    u = lax.dot_general(xb, wu, (((1,), (1,)), ((), ())),
                        preferred_element_type=jnp.float32)
    v = lax.dot_general(xb, wv, (((1,), (1,)), ((), ())),
                        preferred_element_type=jnp.float32)
    g = u * (v * jax.nn.sigmoid(v))
    return lax.dot_general(g, wo, (((1,), (1,)), ((), ())),
                           preferred_element_type=jnp.float32)


def _gmm_body(e_ref, v_ref, pm_ref, xs_ref, wu0_ref, wv0_ref, wo0_ref,
              wu1_ref, wv1_ref, wo1_ref, wu2_ref, wv2_ref, wo2_ref, ys_ref):
    t = pl.program_id(0)
    wrefs = [(wu0_ref, wv0_ref, wo0_ref), (wu1_ref, wv1_ref, wo1_ref),
             (wu2_ref, wv2_ref, wo2_ref)]

    def run(n):
        ys = [_swiglu_tile(xs_ref[j * BT:(j + 1) * BT, :], *[w[0] for w in
                           wrefs[j]]) for j in range(n)]
        for j, y in enumerate(ys):
            ys_ref[j * BT:(j + 1) * BT, :] = y

    @pl.when(v_ref[GT * t + 2] == 1)
    def _():
        run(3)

    @pl.when((v_ref[GT * t + 1] == 1) & (v_ref[GT * t + 2] == 0))
    def _():
        run(2)

    @pl.when((v_ref[GT * t] == 1) & (v_ref[GT * t + 1] == 0))
    def _():
        run(1)


def _gmm(xs, Win, Wout, e_of_t, valid_t, pmap):
    wspecs = []
    for j in range(GT):
        wspecs += [
            pl.BlockSpec((1, F, D),
                         lambda t, e, v, pm, j=j: (e[GT * t + j], 0, 0)),
            pl.BlockSpec((1, F, D),
                         lambda t, e, v, pm, j=j: (e[GT * t + j], 1, 0)),
            pl.BlockSpec((1, D, F),
                         lambda t, e, v, pm, j=j: (e[GT * t + j], 0, 0)),
        ]
    grid_spec = pltpu.PrefetchScalarGridSpec(
        num_scalar_prefetch=3,
        grid=(NTG,),
        in_specs=[
            pl.BlockSpec((GT * BT, D), lambda t, e, v, pm: (pm[t], 0)),
        ] + wspecs,
        out_specs=pl.BlockSpec((GT * BT, D), lambda t, e, v, pm: (pm[t], 0)),
    )
    return pl.pallas_call(
        _gmm_body,
        grid_spec=grid_spec,
        out_shape=jax.ShapeDtypeStruct((PR, D), jnp.float32),
        compiler_params=pltpu.CompilerParams(
            dimension_semantics=("arbitrary",),
            vmem_limit_bytes=100 * 2**20,
        ),
    )(e_of_t, valid_t, pmap, xs, Win, Win, Wout, Win, Win, Wout,
      Win, Win, Wout)


def _add_body(a_ref, b_ref, pa_ref, pb_ref, o_ref):
    o_ref[...] = a_ref[...] * pa_ref[...] + b_ref[...] * pb_ref[...]


def _pair_add(parts, p1, p2):
    BO = 256
    return pl.pallas_call(
        _add_body,
        grid=(T // BO,),
        in_specs=[
            pl.BlockSpec((BO, D), lambda t: (t, 0)),
            pl.BlockSpec((BO, D), lambda t: (t + T // BO, 0)),
            pl.BlockSpec((BO, 1), lambda t: (t, 0)),
            pl.BlockSpec((BO, 1), lambda t: (t, 0)),
        ],
        out_specs=pl.BlockSpec((BO, D), lambda t: (t, 0)),
        out_shape=jax.ShapeDtypeStruct((T, D), jnp.float32),
    )(parts, parts, p1, p2)


def kernel(x, Wg, Win, Wout):
    x_flat = x.reshape(T, D)
    i1, i2, p1, p2 = _gate(x_flat, Wg)

    e_flat = jnp.concatenate([i1, i2], axis=1).reshape(-1)
    p_flat = jnp.concatenate([p1, p2], axis=1).reshape(-1)

    oh = (e_flat[:, None] == jnp.arange(E, dtype=jnp.int32)[None, :])
    ohi = oh.astype(jnp.int32)
    rank = jnp.take_along_axis(jnp.cumsum(ohi, axis=0), e_flat[:, None],
                               axis=1)[:, 0] - 1
    counts = jnp.sum(ohi, axis=0)
    ptiles = (counts + BT - 1) // BT
    tstart = jnp.concatenate(
        [jnp.zeros(1, jnp.int32), jnp.cumsum(ptiles)]).astype(jnp.int32)
    total_tiles = tstart[E]
    pstart = tstart[:E] * BT
    ppos = pstart[e_flat] + rank

    tok_of_a = jnp.arange(A, dtype=jnp.int32) // K
    gidx = (jnp.arange(PR, dtype=jnp.int32) % T).at[ppos].set(tok_of_a)
    srcall = ppos.reshape(T, K).T.reshape(A)

    t_ar = jnp.arange(NT, dtype=jnp.int32)
    raw = (jnp.searchsorted(tstart, t_ar, side="right") - 1).astype(jnp.int32)
    raw = jnp.clip(raw, 0, E - 1)
    e_last = raw[jnp.clip(total_tiles - 1, 0, NT - 1)]
    e_of_t = jnp.where(t_ar < total_tiles, raw, e_last)
    valid_t = (t_ar < total_tiles).astype(jnp.int32)
    pmap = jnp.minimum(jnp.arange(NTG, dtype=jnp.int32),
                       (total_tiles - 1) // GT)

    xs = _sc_gather(x_flat, gidx, PR)
    ys = _gmm(xs, Win, Wout, e_of_t, valid_t, pmap)
    parts = _sc_gather(ys, srcall, A)
    out = _pair_add(parts, p1, p2)
    return out.reshape(1, T, D)


# speedup vs baseline: 1.0214x; 1.0214x over previous
import functools

import jax
import jax.numpy as jnp
from jax import lax
from jax.experimental import pallas as pl
from jax.experimental.pallas import tpu as pltpu
from jax.experimental.pallas import tpu_sc as plsc

E = 64
K = 2
T = 2048
D = 768
F = 768
A = T * K
BT = 128
NT = A // BT + E
GT = 2
NTG = NT // GT
PR = NT * BT

NC = 2
NS = 16
NW = NC * NS


def _gate_body(x_ref, wg_ref, i1_ref, i2_ref, p1_ref, p2_ref):
    s = lax.dot_general(x_ref[...], wg_ref[...], (((1,), (1,)), ((), ())),
                        preferred_element_type=jnp.float32)
    bg = s.shape[0]
    lane = lax.broadcasted_iota(jnp.int32, (bg, E), 1)
    m1 = jnp.max(s, axis=1, keepdims=True)
    i1 = jnp.min(jnp.where(s >= m1, lane, E), axis=1, keepdims=True)
    s2 = jnp.where(lane == i1, -jnp.inf, s)
    m2 = jnp.max(s2, axis=1, keepdims=True)
    i2 = jnp.min(jnp.where(s2 >= m2, lane, E), axis=1, keepdims=True)
    p1 = jax.nn.sigmoid(m1 - m2)
    i1_ref[...] = i1
    i2_ref[...] = i2
    p1_ref[...] = p1
    p2_ref[...] = 1.0 - p1


def _gate(x_flat, Wg):
    BG = 256
    out = jax.ShapeDtypeStruct((T, 1), jnp.int32)
    outf = jax.ShapeDtypeStruct((T, 1), jnp.float32)
    return pl.pallas_call(
        _gate_body,
        grid=(T // BG,),
        in_specs=[
            pl.BlockSpec((BG, D), lambda t: (t, 0)),
            pl.BlockSpec((E, D), lambda t: (0, 0)),
        ],
        out_specs=[pl.BlockSpec((BG, 1), lambda t: (t, 0))] * 4,
        out_shape=[out, out, outf, outf],
    )(x_flat, Wg)


def _sc_gather_body(rw, c, table_hbm, idx_hbm, out_hbm, idx_v, rows_v, sem):
    wid = lax.axis_index("s") * NC + lax.axis_index("c")
    base = wid * rw

    def chunk(j, _):
        off = base + j * c
        pltpu.sync_copy(idx_hbm.at[pl.ds(off, c)], idx_v)
        pltpu.async_copy(table_hbm.at[idx_v], rows_v, sem).wait()
        pltpu.sync_copy(rows_v, out_hbm.at[pl.ds(off, c)])
        return 0

    lax.fori_loop(0, rw // c, chunk, 0)


def _sc_gather_dyn_body(c, table_hbm, idx_hbm, kvec_hbm, out_hbm,
                        idx_v, rows_v, kv, sem):
    wid = lax.axis_index("s") * NC + lax.axis_index("c")
    pltpu.sync_copy(kvec_hbm, kv)
    k = jnp.max(kv[...]).astype(jnp.int32)
    base = wid * (k * c)

    def chunk(j, _):
        off = base + j * c
        pltpu.sync_copy(idx_hbm.at[pl.ds(off, c)], idx_v)
        pltpu.async_copy(table_hbm.at[idx_v], rows_v, sem).wait()
        pltpu.sync_copy(rows_v, out_hbm.at[pl.ds(off, c)])
        return 0

    lax.fori_loop(0, k, chunk, 0)


def _sc_gather(table, idx, n_rows, chunk=128, kvec=None):
    rw = n_rows // NW
    mesh = plsc.VectorSubcoreMesh(core_axis_name="c", subcore_axis_name="s")
    scratch = [
        pltpu.VMEM((chunk,), jnp.int32),
        pltpu.VMEM((chunk, D), jnp.float32),
    ]
    if kvec is None:
        kern = pl.kernel(
            functools.partial(_sc_gather_body, rw, chunk),
            out_type=jax.ShapeDtypeStruct((n_rows, D), jnp.float32),
            mesh=mesh,
            scratch_types=scratch + [pltpu.SemaphoreType.DMA],
        )
        return kern(table, idx)
    kern = pl.kernel(
        functools.partial(_sc_gather_dyn_body, chunk),
        out_type=jax.ShapeDtypeStruct((n_rows, D), jnp.float32),
        mesh=mesh,
        scratch_types=scratch + [pltpu.VMEM((16,), jnp.float32),
                                 pltpu.SemaphoreType.DMA],
        compiler_params=pltpu.CompilerParams(needs_layout_passes=False),
    )
    return kern(table, idx, kvec)


def _swiglu_tile(xb, wu, wv, wo):
    u = lax.dot_general(xb, wu, (((1,), (1,)), ((), ())),
                        preferred_element_type=jnp.float32)
    v = lax.dot_general(xb, wv, (((1,), (1,)), ((), ())),
                        preferred_element_type=jnp.float32)
    g = u * (v * jax.nn.sigmoid(v))
    return lax.dot_general(g, wo, (((1,), (1,)), ((), ())),
                           preferred_element_type=jnp.float32)


def _gmm_body(e_ref, v_ref, pm_ref, xs_ref, wu0_ref, wv0_ref, wo0_ref,
              wu1_ref, wv1_ref, wo1_ref, ys_ref):
    t = pl.program_id(0)
    wrefs = [(wu0_ref, wv0_ref, wo0_ref), (wu1_ref, wv1_ref, wo1_ref)]

    def run(n):
        ys = [_swiglu_tile(xs_ref[j * BT:(j + 1) * BT, :], *[w[0] for w in
                           wrefs[j]]) for j in range(n)]
        for j, y in enumerate(ys):
            ys_ref[j * BT:(j + 1) * BT, :] = y

    @pl.when(v_ref[GT * t + 1] == 1)
    def _():
        run(2)

    @pl.when((v_ref[GT * t] == 1) & (v_ref[GT * t + 1] == 0))
    def _():
        run(1)


def _gmm(xs, Win, Wout, e_of_t, valid_t, pmap):
    wspecs = []
    for j in range(GT):
        wspecs += [
            pl.BlockSpec((1, F, D),
                         lambda t, e, v, pm, j=j: (e[GT * t + j], 0, 0)),
            pl.BlockSpec((1, F, D),
                         lambda t, e, v, pm, j=j: (e[GT * t + j], 1, 0)),
            pl.BlockSpec((1, D, F),
                         lambda t, e, v, pm, j=j: (e[GT * t + j], 0, 0)),
        ]
    grid_spec = pltpu.PrefetchScalarGridSpec(
        num_scalar_prefetch=3,
        grid=(NTG,),
        in_specs=[
            pl.BlockSpec((GT * BT, D), lambda t, e, v, pm: (pm[t], 0)),
        ] + wspecs,
        out_specs=pl.BlockSpec((GT * BT, D), lambda t, e, v, pm: (pm[t], 0)),
    )
    return pl.pallas_call(
        _gmm_body,
        grid_spec=grid_spec,
        out_shape=jax.ShapeDtypeStruct((PR, D), jnp.float32),
        compiler_params=pltpu.CompilerParams(
            dimension_semantics=("arbitrary",),
            vmem_limit_bytes=100 * 2**20,
        ),
    )(e_of_t, valid_t, pmap, xs, Win, Win, Wout, Win, Win, Wout)


def _add_body(a_ref, b_ref, pa_ref, pb_ref, o_ref):
    o_ref[...] = a_ref[...] * pa_ref[...] + b_ref[...] * pb_ref[...]


def _pair_add(parts, p1, p2):
    BO = 256
    return pl.pallas_call(
        _add_body,
        grid=(T // BO,),
        in_specs=[
            pl.BlockSpec((BO, D), lambda t: (t, 0)),
            pl.BlockSpec((BO, D), lambda t: (t + T // BO, 0)),
            pl.BlockSpec((BO, 1), lambda t: (t, 0)),
            pl.BlockSpec((BO, 1), lambda t: (t, 0)),
        ],
        out_specs=pl.BlockSpec((BO, D), lambda t: (t, 0)),
        out_shape=jax.ShapeDtypeStruct((T, D), jnp.float32),
    )(parts, parts, p1, p2)


def kernel(x, Wg, Win, Wout):
    x_flat = x.reshape(T, D)
    i1, i2, p1, p2 = _gate(x_flat, Wg)

    e_flat = jnp.concatenate([i1, i2], axis=1).reshape(-1)
    p_flat = jnp.concatenate([p1, p2], axis=1).reshape(-1)

    oh = (e_flat[:, None] == jnp.arange(E, dtype=jnp.int32)[None, :])
    ohi = oh.astype(jnp.int32)
    rank = jnp.take_along_axis(jnp.cumsum(ohi, axis=0), e_flat[:, None],
                               axis=1)[:, 0] - 1
    counts = jnp.sum(ohi, axis=0)
    ptiles = (counts + BT - 1) // BT
    tstart = jnp.concatenate(
        [jnp.zeros(1, jnp.int32), jnp.cumsum(ptiles)]).astype(jnp.int32)
    total_tiles = tstart[E]
    pstart = tstart[:E] * BT
    ppos = pstart[e_flat] + rank

    tok_of_a = jnp.arange(A, dtype=jnp.int32) // K
    gidx = (jnp.arange(PR, dtype=jnp.int32) % T).at[ppos].set(tok_of_a)
    srcall = ppos.reshape(T, K).T.reshape(A)

    t_ar = jnp.arange(NT, dtype=jnp.int32)
    raw = (jnp.searchsorted(tstart, t_ar, side="right") - 1).astype(jnp.int32)
    raw = jnp.clip(raw, 0, E - 1)
    e_last = raw[jnp.clip(total_tiles - 1, 0, NT - 1)]
    e_of_t = jnp.where(t_ar < total_tiles, raw, e_last)
    valid_t = (t_ar < total_tiles).astype(jnp.int32)
    pmap = jnp.minimum(jnp.arange(NTG, dtype=jnp.int32),
                       (total_tiles - 1) // GT)

    kvec = jnp.full((16,), (total_tiles + NW - 1) // NW, jnp.float32)
    xs = _sc_gather(x_flat, gidx, PR, kvec=kvec)
    ys = _gmm(xs, Win, Wout, e_of_t, valid_t, pmap)
    parts = _sc_gather(ys, srcall, A)
    out = _pair_add(parts, p1, p2)
    return out.reshape(1, T, D)
